# baseline (speedup 1.0000x reference)
import functools

import jax
import jax.numpy as jnp
from jax import lax
from jax.experimental import pallas as pl
from jax.experimental.pallas import tpu as pltpu
from jax.experimental.pallas import tpu_sc as plsc

D_MODEL = 768
EPS = 1e-9
L = 16
NCOL = D_MODEL // L
NUM_CORES = 2
NUM_SUBCORES = 16
NW = NUM_CORES * NUM_SUBCORES
CHUNK = 64
QUAD = 4


def _rsqrt(x):
    one = jnp.full((L,), 1.0, jnp.float32)
    y = jnp.minimum(one, one / x)
    half_x = 0.5 * x
    for _ in range(6):
        y = y * (1.5 - half_x * y * y)
    return y


_GDN = lax.GatherDimensionNumbers(
    offset_dims=(), collapsed_slice_dims=(0,), start_index_map=(0,))


def _shuffle(v, idx):
    return lax.gather(v, idx[:, None], _GDN, (1,),
                      mode=lax.GatherScatterMode.PROMISE_IN_BOUNDS)


def _lane_sum(v, perms):
    for idx in perms:
        v = v + _shuffle(v, idx)
    return v


def _tec_body(bpw, table_hbm, idx_hbm, gamma_hbm, beta_hbm, out_hbm,
              idx_v, rows0, rows1, gamma_v, beta_v,
              means_v, rstds_v,
              gsem0, gsem1, wsem0, wsem1):
    c = lax.axis_index("c")
    s = lax.axis_index("s")
    wid = s * NUM_CORES + c
    base = wid * bpw
    pltpu.sync_copy(idx_hbm.at[pl.ds(base, bpw)], idx_v)
    pltpu.sync_copy(gamma_hbm, gamma_v)
    pltpu.sync_copy(beta_hbm, beta_v)

    nchunk = bpw // CHUNK
    inv_d = jnp.float32(1.0 / D_MODEL)
    lanes = lax.iota(jnp.int32, L)

    bufs = [rows0, rows1]
    gsems = [gsem0, gsem1]
    wsems = [wsem0, wsem1]

    perms = [lanes ^ sh for sh in (1, 2, 4, 8)]

    def compute(rows_v):
        @plsc.parallel_loop(0, CHUNK // QUAD, unroll=2)
        def quad_body(q):
            r0 = q * QUAD
            stats = []
            for rr in range(QUAD):
                acc = jnp.zeros((L,), jnp.float32)
                acc2 = jnp.zeros((L,), jnp.float32)
                for j in range(NCOL):
                    v = rows_v[r0 + rr, pl.ds(j * L, L)]
                    acc = acc + v
                    acc2 = acc2 + v * v
                stats.append((acc, acc2))
            for rr, (acc, acc2) in enumerate(stats):
                mean = _lane_sum(acc, perms) * inv_d
                e2 = _lane_sum(acc2, perms) * inv_d
                rstd = _rsqrt(e2 - mean * mean + jnp.float32(EPS))
                means_v[r0 + rr, pl.ds(0, L)] = mean * rstd
                rstds_v[r0 + rr, pl.ds(0, L)] = rstd

        third = NCOL // 3
        for t in range(3):
            c0 = t * third * L
            gs = [gamma_v[pl.ds(c0 + j * L, L)] for j in range(third)]
            bs = [beta_v[pl.ds(c0 + j * L, L)] for j in range(third)]

            @plsc.parallel_loop(0, CHUNK, unroll=2)
            def row_body(r):
                ms = means_v[r, pl.ds(0, L)]
                sc = rstds_v[r, pl.ds(0, L)]
                for j in range(third):
                    v = rows_v[r, pl.ds(c0 + j * L, L)]
                    rows_v[r, pl.ds(c0 + j * L, L)] = (v * sc - ms) * gs[j] + bs[j]

    gh = [None, None]
    wh = [None, None]
    gh[0] = pltpu.async_copy(
        table_hbm.at[idx_v.at[pl.ds(0, CHUNK)]], bufs[0], gsems[0])
    for ch in range(nchunk):
        cur = ch % 2
        nxt = (ch + 1) % 2
        if ch + 1 < nchunk:
            if wh[nxt] is not None:
                wh[nxt].wait()
            gh[nxt] = pltpu.async_copy(
                table_hbm.at[idx_v.at[pl.ds((ch + 1) * CHUNK, CHUNK)]],
                bufs[nxt], gsems[nxt])
        gh[cur].wait()
        compute(bufs[cur])
        wh[cur] = pltpu.async_copy(
            bufs[cur], out_hbm.at[pl.ds(base + ch * CHUNK, CHUNK)],
            wsems[cur])
    for h in wh:
        if h is not None:
            h.wait()


@functools.partial(jax.jit, static_argnums=(4,))
def _run(weight, idx, gamma, beta, batch_tokens):
    bpw = batch_tokens // NW
    mesh = plsc.VectorSubcoreMesh(
        core_axis_name="c", subcore_axis_name="s",
        num_cores=NUM_CORES, num_subcores=NUM_SUBCORES,
    )
    body = functools.partial(_tec_body, bpw)
    return pl.kernel(
        body,
        out_type=jax.ShapeDtypeStruct((batch_tokens, D_MODEL), jnp.float32),
        mesh=mesh,
        scratch_types=[
            pltpu.VMEM((bpw,), jnp.int32),
            pltpu.VMEM((CHUNK, D_MODEL), jnp.float32),
            pltpu.VMEM((CHUNK, D_MODEL), jnp.float32),
            pltpu.VMEM((D_MODEL,), jnp.float32),
            pltpu.VMEM((D_MODEL,), jnp.float32),
            pltpu.VMEM((CHUNK, L), jnp.float32),
            pltpu.VMEM((CHUNK, L), jnp.float32),
            pltpu.SemaphoreType.DMA,
            pltpu.SemaphoreType.DMA,
            pltpu.SemaphoreType.DMA,
            pltpu.SemaphoreType.DMA,
        ],
    )(weight, idx, gamma, beta)


def kernel(input_ids, weight, ln_gamma, ln_beta):
    b, s = input_ids.shape
    idx = input_ids.reshape(-1).astype(jnp.int32)
    out = _run(weight, idx, ln_gamma, ln_beta, b * s)
    return out.reshape(b, s, D_MODEL)


# speedup vs baseline: 1.0420x; 1.0420x over previous
import functools

import jax
import jax.numpy as jnp
from jax import lax
from jax.experimental import pallas as pl
from jax.experimental.pallas import tpu as pltpu
from jax.experimental.pallas import tpu_sc as plsc

D_MODEL = 768
EPS = 1e-9
L = 16
NCOL = D_MODEL // L
NUM_CORES = 2
NUM_SUBCORES = 16
NW = NUM_CORES * NUM_SUBCORES
CHUNK = 64
QUAD = 4


def _rsqrt(x):
    one = jnp.full((L,), 1.0, jnp.float32)
    y = jnp.minimum(one, one / x)
    half_x = 0.5 * x
    for _ in range(6):
        y = y * (1.5 - half_x * y * y)
    return y


_GDN = lax.GatherDimensionNumbers(
    offset_dims=(), collapsed_slice_dims=(0,), start_index_map=(0,))


def _shuffle(v, idx):
    return lax.gather(v, idx[:, None], _GDN, (1,),
                      mode=lax.GatherScatterMode.PROMISE_IN_BOUNDS)


def _lane_sum(v, perms):
    for idx in perms:
        v = v + _shuffle(v, idx)
    return v


def _tec_body(bpw, table_hbm, idx_hbm, gamma_hbm, beta_hbm, out_hbm,
              idx_v, rows0, rows1, gamma_v, beta_v,
              means_v, rstds_v,
              gsem0, gsem1, wsem0, wsem1, ssem):
    c = lax.axis_index("c")
    s = lax.axis_index("s")
    wid = s * NUM_CORES + c
    base = wid * bpw
    ih = pltpu.async_copy(idx_hbm.at[pl.ds(base, bpw)], idx_v, ssem)
    gh_g = pltpu.async_copy(gamma_hbm, gamma_v, wsem0)
    gh_b = pltpu.async_copy(beta_hbm, beta_v, wsem1)
    ih.wait()

    nchunk = bpw // CHUNK
    inv_d = jnp.float32(1.0 / D_MODEL)
    lanes = lax.iota(jnp.int32, L)

    bufs = [rows0, rows1]
    gsems = [gsem0, gsem1]
    wsems = [wsem0, wsem1]

    perms = [lanes ^ sh for sh in (1, 2, 4, 8)]

    def compute(rows_v):
        @plsc.parallel_loop(0, CHUNK // QUAD, unroll=2)
        def quad_body(q):
            r0 = q * QUAD
            stats = []
            for rr in range(QUAD):
                acc = jnp.zeros((L,), jnp.float32)
                acc2 = jnp.zeros((L,), jnp.float32)
                for j in range(NCOL):
                    v = rows_v[r0 + rr, pl.ds(j * L, L)]
                    acc = acc + v
                    acc2 = acc2 + v * v
                stats.append((acc, acc2))
            for rr, (acc, acc2) in enumerate(stats):
                mean = _lane_sum(acc, perms) * inv_d
                e2 = _lane_sum(acc2, perms) * inv_d
                rstd = _rsqrt(e2 - mean * mean + jnp.float32(EPS))
                means_v[r0 + rr, pl.ds(0, L)] = mean
                rstds_v[r0 + rr, pl.ds(0, L)] = rstd

        third = NCOL // 3
        for t in range(3):
            c0 = t * third * L
            gs = [gamma_v[pl.ds(c0 + j * L, L)] for j in range(third)]
            bs = [beta_v[pl.ds(c0 + j * L, L)] for j in range(third)]

            @plsc.parallel_loop(0, CHUNK, unroll=2)
            def row_body(r):
                m = means_v[r, pl.ds(0, L)]
                sc = rstds_v[r, pl.ds(0, L)]
                for j in range(third):
                    v = rows_v[r, pl.ds(c0 + j * L, L)]
                    rows_v[r, pl.ds(c0 + j * L, L)] = (v - m) * sc * gs[j] + bs[j]

    gh = [None, None]
    wh = [None, None]
    gh[0] = pltpu.async_copy(
        table_hbm.at[idx_v.at[pl.ds(0, CHUNK)]], bufs[0], gsems[0])
    for ch in range(nchunk):
        cur = ch % 2
        nxt = (ch + 1) % 2
        if ch + 1 < nchunk:
            if wh[nxt] is not None:
                wh[nxt].wait()
            gh[nxt] = pltpu.async_copy(
                table_hbm.at[idx_v.at[pl.ds((ch + 1) * CHUNK, CHUNK)]],
                bufs[nxt], gsems[nxt])
        gh[cur].wait()
        if ch == 0:
            gh_g.wait()
            gh_b.wait()
        compute(bufs[cur])
        wh[cur] = pltpu.async_copy(
            bufs[cur], out_hbm.at[pl.ds(base + ch * CHUNK, CHUNK)],
            wsems[cur])
    for h in wh:
        if h is not None:
            h.wait()


@functools.partial(jax.jit, static_argnums=(4,))
def _run(weight, idx, gamma, beta, batch_tokens):
    bpw = batch_tokens // NW
    mesh = plsc.VectorSubcoreMesh(
        core_axis_name="c", subcore_axis_name="s",
        num_cores=NUM_CORES, num_subcores=NUM_SUBCORES,
    )
    body = functools.partial(_tec_body, bpw)
    return pl.kernel(
        body,
        out_type=jax.ShapeDtypeStruct((batch_tokens, D_MODEL), jnp.float32),
        mesh=mesh,
        scratch_types=[
            pltpu.VMEM((bpw,), jnp.int32),
            pltpu.VMEM((CHUNK, D_MODEL), jnp.float32),
            pltpu.VMEM((CHUNK, D_MODEL), jnp.float32),
            pltpu.VMEM((D_MODEL,), jnp.float32),
            pltpu.VMEM((D_MODEL,), jnp.float32),
            pltpu.VMEM((CHUNK, L), jnp.float32),
            pltpu.VMEM((CHUNK, L), jnp.float32),
            pltpu.SemaphoreType.DMA,
            pltpu.SemaphoreType.DMA,
            pltpu.SemaphoreType.DMA,
            pltpu.SemaphoreType.DMA,
            pltpu.SemaphoreType.DMA,
        ],
    )(weight, idx, gamma, beta)


def kernel(input_ids, weight, ln_gamma, ln_beta):
    b, s = input_ids.shape
    idx = input_ids.reshape(-1).astype(jnp.int32)
    out = _run(weight, idx, ln_gamma, ln_beta, b * s)
    return out.reshape(b, s, D_MODEL)


# speedup vs baseline: 1.0579x; 1.0153x over previous
import functools

import jax
import jax.numpy as jnp
from jax import lax
from jax.experimental import pallas as pl
from jax.experimental.pallas import tpu as pltpu
from jax.experimental.pallas import tpu_sc as plsc

D_MODEL = 768
EPS = 1e-9
L = 16
NCOL = D_MODEL // L
NUM_CORES = 2
NUM_SUBCORES = 16
NW = NUM_CORES * NUM_SUBCORES
CHUNK = 64
QUAD = 4


def _rsqrt(x):
    one = jnp.full((L,), 1.0, jnp.float32)
    y = jnp.minimum(one, one / x)
    half_x = 0.5 * x
    for _ in range(6):
        y = y * (1.5 - half_x * y * y)
    return y


_GDN = lax.GatherDimensionNumbers(
    offset_dims=(), collapsed_slice_dims=(0,), start_index_map=(0,))


def _shuffle(v, idx):
    return lax.gather(v, idx[:, None], _GDN, (1,),
                      mode=lax.GatherScatterMode.PROMISE_IN_BOUNDS)


def _lane_sum(v, perms):
    for idx in perms:
        v = v + _shuffle(v, idx)
    return v


def _tec_body(bpw, table_hbm, idx_hbm, gamma_hbm, beta_hbm, out_hbm,
              idx_v, rows0, rows1, gamma_v, beta_v,
              means_v, rstds_v,
              gsem0, gsem1, wsem0, wsem1, ssem):
    c = lax.axis_index("c")
    s = lax.axis_index("s")
    wid = s * NUM_CORES + c
    base = wid * bpw
    ih = pltpu.async_copy(idx_hbm.at[pl.ds(base, bpw)], idx_v, ssem)
    gh_g = pltpu.async_copy(gamma_hbm, gamma_v, wsem0)
    gh_b = pltpu.async_copy(beta_hbm, beta_v, wsem1)
    ih.wait()

    nchunk = bpw // CHUNK
    inv_d = jnp.float32(1.0 / D_MODEL)
    lanes = lax.iota(jnp.int32, L)

    bufs = [rows0, rows1]
    gsems = [gsem0, gsem1]
    wsems = [wsem0, wsem1]

    perms = [lanes ^ sh for sh in (1, 2, 4, 8)]

    def compute(rows_v, nrows):
        @plsc.parallel_loop(0, nrows // QUAD, unroll=2)
        def quad_body(q):
            r0 = q * QUAD
            stats = []
            for rr in range(QUAD):
                acc = jnp.zeros((L,), jnp.float32)
                acc2 = jnp.zeros((L,), jnp.float32)
                for j in range(NCOL):
                    v = rows_v[r0 + rr, pl.ds(j * L, L)]
                    acc = acc + v
                    acc2 = acc2 + v * v
                stats.append((acc, acc2))
            for rr, (acc, acc2) in enumerate(stats):
                mean = _lane_sum(acc, perms) * inv_d
                e2 = _lane_sum(acc2, perms) * inv_d
                rstd = _rsqrt(e2 - mean * mean + jnp.float32(EPS))
                means_v[r0 + rr, pl.ds(0, L)] = mean
                rstds_v[r0 + rr, pl.ds(0, L)] = rstd

        third = NCOL // 3
        for t in range(3):
            c0 = t * third * L
            gs = [gamma_v[pl.ds(c0 + j * L, L)] for j in range(third)]
            bs = [beta_v[pl.ds(c0 + j * L, L)] for j in range(third)]

            @plsc.parallel_loop(0, nrows, unroll=2)
            def row_body(r):
                m = means_v[r, pl.ds(0, L)]
                sc = rstds_v[r, pl.ds(0, L)]
                for j in range(third):
                    v = rows_v[r, pl.ds(c0 + j * L, L)]
                    rows_v[r, pl.ds(c0 + j * L, L)] = (v - m) * sc * gs[j] + bs[j]

    sizes = [16, 48] + [CHUNK] * (nchunk - 1)
    offs = [0]
    for sz in sizes[:-1]:
        offs.append(offs[-1] + sz)
    nsteps = len(sizes)
    gh = [None, None]
    wh = [None, None]
    gh[0] = pltpu.async_copy(
        table_hbm.at[idx_v.at[pl.ds(0, sizes[0])]],
        bufs[0].at[pl.ds(0, sizes[0])], gsems[0])
    for ch in range(nsteps):
        cur = ch % 2
        nxt = (ch + 1) % 2
        if ch + 1 < nsteps:
            if wh[nxt] is not None:
                wh[nxt].wait()
            gh[nxt] = pltpu.async_copy(
                table_hbm.at[idx_v.at[pl.ds(offs[ch + 1], sizes[ch + 1])]],
                bufs[nxt].at[pl.ds(0, sizes[ch + 1])], gsems[nxt])
        gh[cur].wait()
        if ch == 0:
            gh_g.wait()
            gh_b.wait()
        compute(bufs[cur], sizes[ch])
        wh[cur] = pltpu.async_copy(
            bufs[cur].at[pl.ds(0, sizes[ch])],
            out_hbm.at[pl.ds(base + offs[ch], sizes[ch])],
            wsems[cur])
    for h in wh:
        if h is not None:
            h.wait()


@functools.partial(jax.jit, static_argnums=(4,))
def _run(weight, idx, gamma, beta, batch_tokens):
    bpw = batch_tokens // NW
    mesh = plsc.VectorSubcoreMesh(
        core_axis_name="c", subcore_axis_name="s",
        num_cores=NUM_CORES, num_subcores=NUM_SUBCORES,
    )
    body = functools.partial(_tec_body, bpw)
    return pl.kernel(
        body,
        out_type=jax.ShapeDtypeStruct((batch_tokens, D_MODEL), jnp.float32),
        mesh=mesh,
        scratch_types=[
            pltpu.VMEM((bpw,), jnp.int32),
            pltpu.VMEM((CHUNK, D_MODEL), jnp.float32),
            pltpu.VMEM((CHUNK, D_MODEL), jnp.float32),
            pltpu.VMEM((D_MODEL,), jnp.float32),
            pltpu.VMEM((D_MODEL,), jnp.float32),
            pltpu.VMEM((CHUNK, L), jnp.float32),
            pltpu.VMEM((CHUNK, L), jnp.float32),
            pltpu.SemaphoreType.DMA,
            pltpu.SemaphoreType.DMA,
            pltpu.SemaphoreType.DMA,
            pltpu.SemaphoreType.DMA,
            pltpu.SemaphoreType.DMA,
        ],
    )(weight, idx, gamma, beta)


def kernel(input_ids, weight, ln_gamma, ln_beta):
    b, s = input_ids.shape
    idx = input_ids.reshape(-1).astype(jnp.int32)
    out = _run(weight, idx, ln_gamma, ln_beta, b * s)
    return out.reshape(b, s, D_MODEL)
